# baseline (speedup 1.0000x reference)
import functools

import jax
import jax.numpy as jnp
import numpy as np
from jax import lax
from jax.experimental import pallas as pl
from jax.experimental.pallas import tpu as pltpu
from jax.experimental.pallas import tpu_sc as plsc

_B, _N, _K = 2, 1024, 32
_D, _H, _P = 128, 128, 8
_SCALE = 10.0
_TW = 128
_CW = 240
_T1 = 256
_TN = 128
_E2 = _TN * _K


def _pass1_body(x12_ref, hv_ref, wp_ref, bp_ref, w1c_ref, wnode_ref, b1_ref,
                wv_ref, table_ref, ctra_ref, z1_ref):
    x = x12_ref[...]
    hv = hv_ref[...]
    xn, xca, xc = x[:, 0:3], x[:, 3:6], x[:, 6:9]
    v1 = xc - xca
    v2 = xn - xca
    e1 = v1 / jnp.sqrt(jnp.sum(v1 * v1, -1, keepdims=True) + 1e-8)
    u2 = v2 - e1 * jnp.sum(e1 * v2, -1, keepdims=True)
    e2 = u2 / jnp.sqrt(jnp.sum(u2 * u2, -1, keepdims=True) + 1e-8)
    a1, a2, a3 = e1[:, 0:1], e1[:, 1:2], e1[:, 2:3]
    c1, c2, c3 = e2[:, 0:1], e2[:, 1:2], e2[:, 2:3]
    e3 = jnp.concatenate([a2 * c3 - a3 * c2, a3 * c1 - a1 * c3,
                          a1 * c2 - a2 * c1], axis=1)
    t = xca / _SCALE
    frames = jnp.concatenate([e1, e2, e3, t], axis=1)
    pl_cm = jnp.dot(hv, wp_ref[...],
                    preferred_element_type=jnp.float32) + bp_ref[...]
    plno = jnp.sqrt(pl_cm[:, 0:8] ** 2 + pl_cm[:, 8:16] ** 2
                    + pl_cm[:, 16:24] ** 2 + 1e-8)
    pgs = []
    for i in range(3):
        s = (frames[:, 0 + i:1 + i] * pl_cm[:, 0:8]
             + frames[:, 3 + i:4 + i] * pl_cm[:, 8:16]
             + frames[:, 6 + i:7 + i] * pl_cm[:, 16:24]
             + frames[:, 9 + i:10 + i])
        pgs.append(s)
    r = jnp.dot(hv, w1c_ref[...], preferred_element_type=jnp.float32)
    ru = lax.bitcast_convert_type(r, jnp.uint32) + jnp.uint32(0x8000)
    hi = ru[:, 0:64] & jnp.uint32(0xFFFF0000)
    lo = (ru[:, 64:128] & jnp.uint32(0xFFFF0000)) >> 16
    rp = lax.bitcast_convert_type(hi | lo, jnp.float32)
    pad = jnp.zeros((x.shape[0], _TW - 88), jnp.float32)
    table_ref[...] = jnp.concatenate([rp] + pgs + [pad], axis=1)
    ones8 = jnp.ones((1, 8), jnp.float32)
    tb = jnp.concatenate([frames[:, 9 + j:10 + j] * ones8 for j in range(3)],
                         axis=1)
    cblk = [jnp.concatenate(
        [frames[:, 3 * i + j:3 * i + j + 1] * ones8 for j in range(3)],
        axis=1) for i in range(3)]
    onesb = jnp.ones_like(tb)
    a_arr = jnp.concatenate([onesb, cblk[0], cblk[1], cblk[2], -onesb], axis=1)
    b_arr = jnp.concatenate(
        [-tb, -tb * cblk[0], -tb * cblk[1], -tb * cblk[2],
         jnp.concatenate(pgs, axis=1)], axis=1)
    ctra_ref[...] = jnp.concatenate([a_arr, b_arr], axis=1)
    del wv_ref
    z1_ref[...] = jnp.dot(jnp.concatenate([hv, pl_cm, plno], axis=1),
                          wnode_ref[...],
                          preferred_element_type=jnp.float32) + b1_ref[...]


def _pass1(x12, hv2, wp_cm, bp_cm, w1c, wnode, b1, wv):
    m = _B * _N
    grid = (m // _T1,)
    row = lambda i: (i, 0)
    whole = lambda i: (0, 0)
    return pl.pallas_call(
        _pass1_body,
        grid=grid,
        in_specs=[
            pl.BlockSpec((_T1, 12), row),
            pl.BlockSpec((_T1, _D), row),
            pl.BlockSpec((_D, 24), whole),
            pl.BlockSpec((1, 24), whole),
            pl.BlockSpec((_D, _H), whole),
            pl.BlockSpec((_D + 32, _H), whole),
            pl.BlockSpec((1, _H), whole),
            pl.BlockSpec((120, _H), whole),
        ],
        out_specs=[
            pl.BlockSpec((_T1, _TW), row),
            pl.BlockSpec((_T1, _CW), row),
            pl.BlockSpec((_T1, _H), row),
        ],
        out_shape=[
            jax.ShapeDtypeStruct((m, _TW), jnp.float32),
            jax.ShapeDtypeStruct((m, _CW), jnp.float32),
            jax.ShapeDtypeStruct((m, _H), jnp.float32),
        ],
    )(x12, hv2, wp_cm, bp_cm, w1c, wnode, b1, wv)


def _sc_gather(table, gidx):
    info = plsc.get_sparse_core_info()
    nw = info.num_cores * info.num_subcores
    tot = gidx.shape[0]
    per_w = tot // nw
    ch = 128
    n_ch = per_w // ch
    mesh = plsc.VectorSubcoreMesh(core_axis_name="c", subcore_axis_name="s")

    @functools.partial(
        pl.kernel, mesh=mesh,
        out_type=jax.ShapeDtypeStruct((tot, _TW), jnp.float32),
        scratch_types=[
            pltpu.VMEM((per_w,), jnp.int32),
            pltpu.VMEM((2, ch, _TW), jnp.float32),
            pltpu.SemaphoreType.DMA,
            pltpu.SemaphoreType.DMA,
        ],
    )
    def k(table_hbm, idx_hbm, out_hbm, idx_v, buf_v, sem0, sem1):
        wid = lax.axis_index("s") * info.num_cores + lax.axis_index("c")
        base = wid * per_w
        pltpu.sync_copy(idx_hbm.at[pl.ds(base, per_w)], idx_v)
        sems = (sem0, sem1)
        cps = [None, None]

        def start(c):
            b = c & 1
            cps[b] = pltpu.async_copy(
                table_hbm.at[idx_v.at[pl.ds(c * ch, ch)]], buf_v.at[b],
                sems[b])

        start(0)
        for c in range(n_ch):
            b = c & 1
            if c + 1 < n_ch:
                start(c + 1)
            cps[b].wait()
            pltpu.sync_copy(buf_v.at[b], out_hbm.at[pl.ds(base + c * ch, ch)])

    return k(table, gidx)


def _ln_rows(x, g, b):
    mu = jnp.mean(x, -1, keepdims=True)
    var = jnp.mean((x - mu) ** 2, -1, keepdims=True)
    return (x - mu) / jnp.sqrt(var + 1e-5) * g + b


def _pass2_body(he_ref, g_ref, ctra_ref, z1_ref, hv_ref,
                exp_ref, trep_ref, ssum_ref,
                w1he_ref, wv_ref, w1nn_ref,
                w2_ref, b2_ref, w3_ref, b3_ref,
                ln0g_ref, ln0b_ref, wd1_ref, bd1_ref, wd2_ref, bd2_ref,
                ln1g_ref, ln1b_ref, out_ref):
    f32 = jnp.float32
    he = he_ref[...]
    g = g_ref[...]
    ctrn = jnp.concatenate([ctra_ref[...], z1_ref[...]], axis=1)
    ab = jnp.dot(exp_ref[...], ctrn.astype(jnp.bfloat16),
                 preferred_element_type=f32)
    gu = lax.bitcast_convert_type(g[:, 0:64], jnp.uint32)
    r_hi = lax.bitcast_convert_type(gu & jnp.uint32(0xFFFF0000), f32)
    r_lo = lax.bitcast_convert_type(gu << 16, f32)
    rfull = jnp.concatenate([r_hi, r_lo], axis=1)
    npg = g[:, 64:88]
    v = (jnp.dot(npg, trep_ref[...], preferred_element_type=f32)
         * ab[:, 0:120] + ab[:, 120:240])
    ns = jnp.dot(v * v, ssum_ref[...], preferred_element_type=f32)
    nn = jnp.sqrt(ns + 1e-8)
    x1 = (ab[:, 240:368] + rfull
          + jnp.dot(he, w1he_ref[...], preferred_element_type=f32)
          + jnp.dot(v, wv_ref[...], preferred_element_type=f32)
          + jnp.dot(nn, w1nn_ref[...], preferred_element_type=f32))
    h1 = jnp.maximum(x1, 0.0)
    h2 = jnp.maximum(
        jnp.dot(h1, w2_ref[...], preferred_element_type=f32) + b2_ref[...], 0.0)
    h3 = jnp.dot(h2, w3_ref[...], preferred_element_type=f32) + b3_ref[...]
    m = jnp.mean(h3.reshape(_TN, _K, _H), axis=1)
    h = _ln_rows(hv_ref[...] + m, ln0g_ref[...], ln0b_ref[...])
    dm = jnp.dot(
        jnp.maximum(jnp.dot(h, wd1_ref[...],
                            preferred_element_type=f32) + bd1_ref[...], 0.0),
        wd2_ref[...], preferred_element_type=f32) + bd2_ref[...]
    out_ref[...] = _ln_rows(h + dm, ln1g_ref[...], ln1b_ref[...])


def _pass2(he_r, g, ctra, z1, hv2, expm, trep, ssum, w1he, wv, w1nn,
           w2, b2, w3, b3, ln0g, ln0b, wd1, bd1, wd2, bd2, ln1g, ln1b,
           base_node, n_nodes):
    grid = (n_nodes // _TN,)
    boff = base_node // _TN
    erow = lambda i: (i + boff, 0)
    nrow = lambda i: (i + boff, 0)
    gorow = lambda i: (i, 0)
    whole = lambda i: (0, 0)
    return pl.pallas_call(
        _pass2_body,
        grid=grid,
        in_specs=[
            pl.BlockSpec((_E2, _D), erow),
            pl.BlockSpec((_E2, _TW), gorow),
            pl.BlockSpec((_TN, _CW), nrow),
            pl.BlockSpec((_TN, _H), nrow),
            pl.BlockSpec((_TN, _D), nrow),
            pl.BlockSpec((_E2, _TN), whole),
            pl.BlockSpec((24, 120), whole),
            pl.BlockSpec((120, 16), whole),
            pl.BlockSpec((_D, _H), whole),
            pl.BlockSpec((120, _H), whole),
            pl.BlockSpec((16, _H), whole),
            pl.BlockSpec((_H, _H), whole),
            pl.BlockSpec((1, _H), whole),
            pl.BlockSpec((_H, _H), whole),
            pl.BlockSpec((1, _H), whole),
            pl.BlockSpec((1, _H), whole),
            pl.BlockSpec((1, _H), whole),
            pl.BlockSpec((_H, 4 * _H), whole),
            pl.BlockSpec((1, 4 * _H), whole),
            pl.BlockSpec((4 * _H, _H), whole),
            pl.BlockSpec((1, _H), whole),
            pl.BlockSpec((1, _H), whole),
            pl.BlockSpec((1, _H), whole),
        ],
        out_specs=pl.BlockSpec((_TN, _H), gorow),
        out_shape=jax.ShapeDtypeStruct((n_nodes, _H), jnp.float32),
    )(he_r, g, ctra, z1, hv2, expm, trep, ssum, w1he, wv, w1nn,
      w2, b2, w3, b3, ln0g, ln0b, wd1, bd1, wd2, bd2, ln1g, ln1b)


def kernel(h_V, h_E, E_idx, X, Wp, bp, W1, b1, W2, b2, W3, b3,
           ln0_g, ln0_b, Wd1, bd1, Wd2, bd2, ln1_g, ln1_b):
    f32 = jnp.float32
    cm = np.array([p * 3 + i for i in range(3) for p in range(_P)])
    wp_cm = Wp[:, cm]
    bp_cm = bp[cm].reshape(1, 24)
    w1c = W1[256:384]
    wnode = jnp.concatenate([W1[0:128], W1[384 + cm], W1[408:416]], axis=0)
    w1nbl = W1[416 + cm]
    w1q = [jnp.concatenate([w1nbl[8 * i:8 * i + 8]] * 3, axis=0)
           for i in range(3)]
    z24 = jnp.zeros((24, _H), f32)
    wv = jnp.concatenate([z24, w1q[0], w1q[1], w1q[2], z24], axis=0)
    w1nn = jnp.concatenate([W1[440:448], W1[448:456]], axis=0)
    w1he = W1[128:256]
    expm = jnp.asarray(np.kron(np.eye(_TN, dtype=np.float32),
                               np.ones((_K, 1), np.float32))
                       ).astype(jnp.bfloat16)
    trep = jnp.asarray(np.tile(np.eye(24, dtype=np.float32), (1, 5)))
    ssum_np = np.zeros((120, 16), np.float32)
    for j in range(3):
        for p in range(8):
            ssum_np[8 * j + p, p] = 1.0
            ssum_np[96 + 8 * j + p, 8 + p] = 1.0
    ssum = jnp.asarray(ssum_np)

    x12 = X.reshape(_B * _N, 12)
    hv2 = h_V.reshape(_B * _N, _D)
    table, ctra, z1 = _pass1(x12, hv2, wp_cm, bp_cm, w1c, wnode,
                             b1.reshape(1, _H), wv)

    gidx = (E_idx + (jnp.arange(_B, dtype=jnp.int32) * _N)[:, None, None]
            ).reshape(-1)
    he_r = h_E.reshape(_B * _N * _K, _D)
    n_chunks = 8
    m = _B * _N
    cn = m // n_chunks
    gs = [_sc_gather(table, lax.slice(gidx, (c * cn * _K,),
                                      ((c + 1) * cn * _K,)))
          for c in range(n_chunks)]
    outs = []
    for c in range(n_chunks):
        outs.append(_pass2(
            he_r, gs[c], ctra, z1, hv2, expm, trep, ssum, w1he, wv, w1nn,
            W2, b2.reshape(1, _H), W3, b3.reshape(1, _H),
            ln0_g.reshape(1, _H), ln0_b.reshape(1, _H),
            Wd1, bd1.reshape(1, 4 * _H), Wd2, bd2.reshape(1, _H),
            ln1_g.reshape(1, _H), ln1_b.reshape(1, _H),
            base_node=c * cn, n_nodes=cn))
    out = jnp.concatenate(outs, axis=0)
    return out.reshape(_B, _N, _H), h_E


# speedup vs baseline: 1.0943x; 1.0943x over previous
import functools

import jax
import jax.numpy as jnp
import numpy as np
from jax import lax
from jax.experimental import pallas as pl
from jax.experimental.pallas import tpu as pltpu
from jax.experimental.pallas import tpu_sc as plsc

_B, _N, _K = 2, 1024, 32
_D, _H, _P = 128, 128, 8
_SCALE = 10.0
_TW = 128
_CW = 240
_T1 = 256
_TN = 128
_E2 = _TN * _K


def _pass1_body(x12_ref, hv_ref, wp_ref, bp_ref, w1c_ref, wnode_ref, b1_ref,
                wv_ref, table_ref, ctra_ref, z1_ref):
    x = x12_ref[...]
    hv = hv_ref[...]
    xn, xca, xc = x[:, 0:3], x[:, 3:6], x[:, 6:9]
    v1 = xc - xca
    v2 = xn - xca
    e1 = v1 / jnp.sqrt(jnp.sum(v1 * v1, -1, keepdims=True) + 1e-8)
    u2 = v2 - e1 * jnp.sum(e1 * v2, -1, keepdims=True)
    e2 = u2 / jnp.sqrt(jnp.sum(u2 * u2, -1, keepdims=True) + 1e-8)
    a1, a2, a3 = e1[:, 0:1], e1[:, 1:2], e1[:, 2:3]
    c1, c2, c3 = e2[:, 0:1], e2[:, 1:2], e2[:, 2:3]
    e3 = jnp.concatenate([a2 * c3 - a3 * c2, a3 * c1 - a1 * c3,
                          a1 * c2 - a2 * c1], axis=1)
    t = xca / _SCALE
    frames = jnp.concatenate([e1, e2, e3, t], axis=1)
    pl_cm = jnp.dot(hv, wp_ref[...],
                    preferred_element_type=jnp.float32) + bp_ref[...]
    plno = jnp.sqrt(pl_cm[:, 0:8] ** 2 + pl_cm[:, 8:16] ** 2
                    + pl_cm[:, 16:24] ** 2 + 1e-8)
    pgs = []
    for i in range(3):
        s = (frames[:, 0 + i:1 + i] * pl_cm[:, 0:8]
             + frames[:, 3 + i:4 + i] * pl_cm[:, 8:16]
             + frames[:, 6 + i:7 + i] * pl_cm[:, 16:24]
             + frames[:, 9 + i:10 + i])
        pgs.append(s)
    r = jnp.dot(hv, w1c_ref[...], preferred_element_type=jnp.float32)
    ru = lax.bitcast_convert_type(r, jnp.uint32) + jnp.uint32(0x8000)
    hi = ru[:, 0:64] & jnp.uint32(0xFFFF0000)
    lo = (ru[:, 64:128] & jnp.uint32(0xFFFF0000)) >> 16
    rp = lax.bitcast_convert_type(hi | lo, jnp.float32)
    pad = jnp.zeros((x.shape[0], _TW - 88), jnp.float32)
    table_ref[...] = jnp.concatenate([rp] + pgs + [pad], axis=1)
    ones8 = jnp.ones((1, 8), jnp.float32)
    tb = jnp.concatenate([frames[:, 9 + j:10 + j] * ones8 for j in range(3)],
                         axis=1)
    cblk = [jnp.concatenate(
        [frames[:, 3 * i + j:3 * i + j + 1] * ones8 for j in range(3)],
        axis=1) for i in range(3)]
    onesb = jnp.ones_like(tb)
    a_arr = jnp.concatenate([onesb, cblk[0], cblk[1], cblk[2], -onesb], axis=1)
    b_arr = jnp.concatenate(
        [-tb, -tb * cblk[0], -tb * cblk[1], -tb * cblk[2],
         jnp.concatenate(pgs, axis=1)], axis=1)
    ctra_ref[...] = jnp.concatenate([a_arr, b_arr], axis=1)
    del wv_ref
    z1_ref[...] = jnp.dot(jnp.concatenate([hv, pl_cm, plno], axis=1),
                          wnode_ref[...],
                          preferred_element_type=jnp.float32) + b1_ref[...]


def _pass1(x12, hv2, wp_cm, bp_cm, w1c, wnode, b1, wv):
    m = _B * _N
    grid = (m // _T1,)
    row = lambda i: (i, 0)
    whole = lambda i: (0, 0)
    return pl.pallas_call(
        _pass1_body,
        grid=grid,
        in_specs=[
            pl.BlockSpec((_T1, 12), row),
            pl.BlockSpec((_T1, _D), row),
            pl.BlockSpec((_D, 24), whole),
            pl.BlockSpec((1, 24), whole),
            pl.BlockSpec((_D, _H), whole),
            pl.BlockSpec((_D + 32, _H), whole),
            pl.BlockSpec((1, _H), whole),
            pl.BlockSpec((120, _H), whole),
        ],
        out_specs=[
            pl.BlockSpec((_T1, _TW), row),
            pl.BlockSpec((_T1, _CW), row),
            pl.BlockSpec((_T1, _H), row),
        ],
        out_shape=[
            jax.ShapeDtypeStruct((m, _TW), jnp.float32),
            jax.ShapeDtypeStruct((m, _CW), jnp.float32),
            jax.ShapeDtypeStruct((m, _H), jnp.float32),
        ],
    )(x12, hv2, wp_cm, bp_cm, w1c, wnode, b1, wv)


def _sc_gather(table, gidx):
    info = plsc.get_sparse_core_info()
    nw = info.num_cores * info.num_subcores
    tot = gidx.shape[0]
    per_w = tot // nw
    ch = 128
    n_ch = per_w // ch
    mesh = plsc.VectorSubcoreMesh(core_axis_name="c", subcore_axis_name="s")

    @functools.partial(
        pl.kernel, mesh=mesh,
        out_type=jax.ShapeDtypeStruct((tot, _TW), jnp.float32),
        scratch_types=[
            pltpu.VMEM((per_w,), jnp.int32),
            pltpu.VMEM((2, ch, _TW), jnp.float32),
            pltpu.SemaphoreType.DMA,
            pltpu.SemaphoreType.DMA,
        ],
    )
    def k(table_hbm, idx_hbm, out_hbm, idx_v, buf_v, sem0, sem1):
        wid = lax.axis_index("s") * info.num_cores + lax.axis_index("c")
        base = wid * per_w
        pltpu.sync_copy(idx_hbm.at[pl.ds(base, per_w)], idx_v)
        sems = (sem0, sem1)
        cps = [None, None]

        def start(c):
            b = c & 1
            cps[b] = pltpu.async_copy(
                table_hbm.at[idx_v.at[pl.ds(c * ch, ch)]], buf_v.at[b],
                sems[b])

        start(0)
        for c in range(n_ch):
            b = c & 1
            if c + 1 < n_ch:
                start(c + 1)
            cps[b].wait()
            pltpu.sync_copy(buf_v.at[b], out_hbm.at[pl.ds(base + c * ch, ch)])

    return k(table, gidx)


def _ln_rows(x, g, b):
    mu = jnp.mean(x, -1, keepdims=True)
    var = jnp.mean((x - mu) ** 2, -1, keepdims=True)
    return (x - mu) / jnp.sqrt(var + 1e-5) * g + b


def _pass2_body(he_ref, g_ref, ctra_ref, z1_ref, hv_ref,
                exp_ref, trep_ref, ssum_ref,
                w1he_ref, wv_ref, w1nn_ref,
                w2_ref, b2_ref, w3_ref, b3_ref,
                ln0g_ref, ln0b_ref, wd1_ref, bd1_ref, wd2_ref, bd2_ref,
                ln1g_ref, ln1b_ref, out_ref):
    f32 = jnp.float32
    he = he_ref[...]
    g = g_ref[...]
    ctrn = jnp.concatenate([ctra_ref[...], z1_ref[...]], axis=1)
    ab = jnp.dot(exp_ref[...], ctrn.astype(jnp.bfloat16),
                 preferred_element_type=f32)
    gu = lax.bitcast_convert_type(g[:, 0:64], jnp.uint32)
    r_hi = lax.bitcast_convert_type(gu & jnp.uint32(0xFFFF0000), f32)
    r_lo = lax.bitcast_convert_type(gu << 16, f32)
    rfull = jnp.concatenate([r_hi, r_lo], axis=1)
    npg = g[:, 64:88]
    v = (jnp.dot(npg, trep_ref[...], preferred_element_type=f32)
         * ab[:, 0:120] + ab[:, 120:240])
    ns = jnp.dot(v * v, ssum_ref[...], preferred_element_type=f32)
    nn = jnp.sqrt(ns + 1e-8)
    x1 = (ab[:, 240:368] + rfull
          + jnp.dot(he, w1he_ref[...], preferred_element_type=f32)
          + jnp.dot(v, wv_ref[...], preferred_element_type=f32)
          + jnp.dot(nn, w1nn_ref[...], preferred_element_type=f32))
    h1 = jnp.maximum(x1, 0.0)
    h2 = jnp.maximum(
        jnp.dot(h1, w2_ref[...], preferred_element_type=f32) + b2_ref[...], 0.0)
    h3 = jnp.dot(h2, w3_ref[...], preferred_element_type=f32) + b3_ref[...]
    m = jnp.mean(h3.reshape(_TN, _K, _H), axis=1)
    h = _ln_rows(hv_ref[...] + m, ln0g_ref[...], ln0b_ref[...])
    dm = jnp.dot(
        jnp.maximum(jnp.dot(h, wd1_ref[...],
                            preferred_element_type=f32) + bd1_ref[...], 0.0),
        wd2_ref[...], preferred_element_type=f32) + bd2_ref[...]
    out_ref[...] = _ln_rows(h + dm, ln1g_ref[...], ln1b_ref[...])


def _pass2(he_r, g, ctra, z1, hv2, expm, trep, ssum, w1he, wv, w1nn,
           w2, b2, w3, b3, ln0g, ln0b, wd1, bd1, wd2, bd2, ln1g, ln1b,
           base_node, n_nodes):
    grid = (n_nodes // _TN,)
    boff = base_node // _TN
    erow = lambda i: (i + boff, 0)
    nrow = lambda i: (i + boff, 0)
    gorow = lambda i: (i, 0)
    whole = lambda i: (0, 0)
    return pl.pallas_call(
        _pass2_body,
        grid=grid,
        in_specs=[
            pl.BlockSpec((_E2, _D), erow),
            pl.BlockSpec((_E2, _TW), gorow),
            pl.BlockSpec((_TN, _CW), nrow),
            pl.BlockSpec((_TN, _H), nrow),
            pl.BlockSpec((_TN, _D), nrow),
            pl.BlockSpec((_E2, _TN), whole),
            pl.BlockSpec((24, 120), whole),
            pl.BlockSpec((120, 16), whole),
            pl.BlockSpec((_D, _H), whole),
            pl.BlockSpec((120, _H), whole),
            pl.BlockSpec((16, _H), whole),
            pl.BlockSpec((_H, _H), whole),
            pl.BlockSpec((1, _H), whole),
            pl.BlockSpec((_H, _H), whole),
            pl.BlockSpec((1, _H), whole),
            pl.BlockSpec((1, _H), whole),
            pl.BlockSpec((1, _H), whole),
            pl.BlockSpec((_H, 4 * _H), whole),
            pl.BlockSpec((1, 4 * _H), whole),
            pl.BlockSpec((4 * _H, _H), whole),
            pl.BlockSpec((1, _H), whole),
            pl.BlockSpec((1, _H), whole),
            pl.BlockSpec((1, _H), whole),
        ],
        out_specs=pl.BlockSpec((_TN, _H), gorow),
        out_shape=jax.ShapeDtypeStruct((n_nodes, _H), jnp.float32),
    )(he_r, g, ctra, z1, hv2, expm, trep, ssum, w1he, wv, w1nn,
      w2, b2, w3, b3, ln0g, ln0b, wd1, bd1, wd2, bd2, ln1g, ln1b)


def kernel(h_V, h_E, E_idx, X, Wp, bp, W1, b1, W2, b2, W3, b3,
           ln0_g, ln0_b, Wd1, bd1, Wd2, bd2, ln1_g, ln1_b):
    f32 = jnp.float32
    cm = np.array([p * 3 + i for i in range(3) for p in range(_P)])
    wp_cm = Wp[:, cm]
    bp_cm = bp[cm].reshape(1, 24)
    w1c = W1[256:384]
    wnode = jnp.concatenate([W1[0:128], W1[384 + cm], W1[408:416]], axis=0)
    w1nbl = W1[416 + cm]
    w1q = [jnp.concatenate([w1nbl[8 * i:8 * i + 8]] * 3, axis=0)
           for i in range(3)]
    z24 = jnp.zeros((24, _H), f32)
    wv = jnp.concatenate([z24, w1q[0], w1q[1], w1q[2], z24], axis=0)
    w1nn = jnp.concatenate([W1[440:448], W1[448:456]], axis=0)
    w1he = W1[128:256]
    expm = jnp.asarray(np.kron(np.eye(_TN, dtype=np.float32),
                               np.ones((_K, 1), np.float32))
                       ).astype(jnp.bfloat16)
    trep = jnp.asarray(np.tile(np.eye(24, dtype=np.float32), (1, 5)))
    ssum_np = np.zeros((120, 16), np.float32)
    for j in range(3):
        for p in range(8):
            ssum_np[8 * j + p, p] = 1.0
            ssum_np[96 + 8 * j + p, 8 + p] = 1.0
    ssum = jnp.asarray(ssum_np)

    x12 = X.reshape(_B * _N, 12)
    hv2 = h_V.reshape(_B * _N, _D)
    table, ctra, z1 = _pass1(x12, hv2, wp_cm, bp_cm, w1c, wnode,
                             b1.reshape(1, _H), wv)

    gidx = (E_idx + (jnp.arange(_B, dtype=jnp.int32) * _N)[:, None, None]
            ).reshape(-1)
    he_r = h_E.reshape(_B * _N * _K, _D)
    n_chunks = 2
    m = _B * _N
    cn = m // n_chunks
    gs = [_sc_gather(table, lax.slice(gidx, (c * cn * _K,),
                                      ((c + 1) * cn * _K,)))
          for c in range(n_chunks)]
    outs = []
    for c in range(n_chunks):
        outs.append(_pass2(
            he_r, gs[c], ctra, z1, hv2, expm, trep, ssum, w1he, wv, w1nn,
            W2, b2.reshape(1, _H), W3, b3.reshape(1, _H),
            ln0_g.reshape(1, _H), ln0_b.reshape(1, _H),
            Wd1, bd1.reshape(1, 4 * _H), Wd2, bd2.reshape(1, _H),
            ln1_g.reshape(1, _H), ln1_b.reshape(1, _H),
            base_node=c * cn, n_nodes=cn))
    out = jnp.concatenate(outs, axis=0)
    return out.reshape(_B, _N, _H), h_E


# speedup vs baseline: 1.1392x; 1.0411x over previous
import functools

import jax
import jax.numpy as jnp
import numpy as np
from jax import lax
from jax.experimental import pallas as pl
from jax.experimental.pallas import tpu as pltpu
from jax.experimental.pallas import tpu_sc as plsc

_B, _N, _K = 2, 1024, 32
_D, _H, _P = 128, 128, 8
_SCALE = 10.0
_TW = 128
_CW = 240
_T1 = 512
_TN = 128
_E2 = _TN * _K


def _pass1_body(x12_ref, hv_ref, wp_ref, bp_ref, w1c_ref, wnode_ref, b1_ref,
                wv_ref, table_ref, ctra_ref, z1_ref):
    x = x12_ref[...]
    hv = hv_ref[...]
    xn, xca, xc = x[:, 0:3], x[:, 3:6], x[:, 6:9]
    v1 = xc - xca
    v2 = xn - xca
    e1 = v1 / jnp.sqrt(jnp.sum(v1 * v1, -1, keepdims=True) + 1e-8)
    u2 = v2 - e1 * jnp.sum(e1 * v2, -1, keepdims=True)
    e2 = u2 / jnp.sqrt(jnp.sum(u2 * u2, -1, keepdims=True) + 1e-8)
    a1, a2, a3 = e1[:, 0:1], e1[:, 1:2], e1[:, 2:3]
    c1, c2, c3 = e2[:, 0:1], e2[:, 1:2], e2[:, 2:3]
    e3 = jnp.concatenate([a2 * c3 - a3 * c2, a3 * c1 - a1 * c3,
                          a1 * c2 - a2 * c1], axis=1)
    t = xca / _SCALE
    frames = jnp.concatenate([e1, e2, e3, t], axis=1)
    pl_cm = jnp.dot(hv, wp_ref[...],
                    preferred_element_type=jnp.float32) + bp_ref[...]
    plno = jnp.sqrt(pl_cm[:, 0:8] ** 2 + pl_cm[:, 8:16] ** 2
                    + pl_cm[:, 16:24] ** 2 + 1e-8)
    pgs = []
    for i in range(3):
        s = (frames[:, 0 + i:1 + i] * pl_cm[:, 0:8]
             + frames[:, 3 + i:4 + i] * pl_cm[:, 8:16]
             + frames[:, 6 + i:7 + i] * pl_cm[:, 16:24]
             + frames[:, 9 + i:10 + i])
        pgs.append(s)
    r = jnp.dot(hv, w1c_ref[...], preferred_element_type=jnp.float32)
    ru = lax.bitcast_convert_type(r, jnp.uint32) + jnp.uint32(0x8000)
    hi = ru[:, 0:64] & jnp.uint32(0xFFFF0000)
    lo = (ru[:, 64:128] & jnp.uint32(0xFFFF0000)) >> 16
    rp = lax.bitcast_convert_type(hi | lo, jnp.float32)
    pad = jnp.zeros((x.shape[0], _TW - 88), jnp.float32)
    table_ref[...] = jnp.concatenate([rp] + pgs + [pad], axis=1)
    ones8 = jnp.ones((1, 8), jnp.float32)
    tb = jnp.concatenate([frames[:, 9 + j:10 + j] * ones8 for j in range(3)],
                         axis=1)
    cblk = [jnp.concatenate(
        [frames[:, 3 * i + j:3 * i + j + 1] * ones8 for j in range(3)],
        axis=1) for i in range(3)]
    onesb = jnp.ones_like(tb)
    a_arr = jnp.concatenate([onesb, cblk[0], cblk[1], cblk[2], -onesb], axis=1)
    b_arr = jnp.concatenate(
        [-tb, -tb * cblk[0], -tb * cblk[1], -tb * cblk[2],
         jnp.concatenate(pgs, axis=1)], axis=1)
    ctra_ref[...] = jnp.concatenate([a_arr, b_arr], axis=1)
    del wv_ref
    z1_ref[...] = jnp.dot(jnp.concatenate([hv, pl_cm, plno], axis=1),
                          wnode_ref[...],
                          preferred_element_type=jnp.float32) + b1_ref[...]


def _pass1(x12, hv2, wp_cm, bp_cm, w1c, wnode, b1, wv):
    m = _B * _N
    grid = (m // _T1,)
    row = lambda i: (i, 0)
    whole = lambda i: (0, 0)
    return pl.pallas_call(
        _pass1_body,
        grid=grid,
        in_specs=[
            pl.BlockSpec((_T1, 12), row),
            pl.BlockSpec((_T1, _D), row),
            pl.BlockSpec((_D, 24), whole),
            pl.BlockSpec((1, 24), whole),
            pl.BlockSpec((_D, _H), whole),
            pl.BlockSpec((_D + 32, _H), whole),
            pl.BlockSpec((1, _H), whole),
            pl.BlockSpec((120, _H), whole),
        ],
        out_specs=[
            pl.BlockSpec((_T1, _TW), row),
            pl.BlockSpec((_T1, _CW), row),
            pl.BlockSpec((_T1, _H), row),
        ],
        out_shape=[
            jax.ShapeDtypeStruct((m, _TW), jnp.float32),
            jax.ShapeDtypeStruct((m, _CW), jnp.float32),
            jax.ShapeDtypeStruct((m, _H), jnp.float32),
        ],
    )(x12, hv2, wp_cm, bp_cm, w1c, wnode, b1, wv)


def _sc_gather(table, gidx):
    info = plsc.get_sparse_core_info()
    nw = info.num_cores * info.num_subcores
    tot = gidx.shape[0]
    per_w = tot // nw
    ch = 128
    n_ch = per_w // ch
    mesh = plsc.VectorSubcoreMesh(core_axis_name="c", subcore_axis_name="s")

    @functools.partial(
        pl.kernel, mesh=mesh,
        out_type=jax.ShapeDtypeStruct((tot, _TW), jnp.float32),
        scratch_types=[
            pltpu.VMEM((per_w,), jnp.int32),
            pltpu.VMEM((2, ch, _TW), jnp.float32),
            pltpu.SemaphoreType.DMA,
            pltpu.SemaphoreType.DMA,
        ],
    )
    def k(table_hbm, idx_hbm, out_hbm, idx_v, buf_v, sem0, sem1):
        wid = lax.axis_index("s") * info.num_cores + lax.axis_index("c")
        base = wid * per_w
        pltpu.sync_copy(idx_hbm.at[pl.ds(base, per_w)], idx_v)
        sems = (sem0, sem1)
        cps = [None, None]

        def start(c):
            b = c & 1
            cps[b] = pltpu.async_copy(
                table_hbm.at[idx_v.at[pl.ds(c * ch, ch)]], buf_v.at[b],
                sems[b])

        start(0)
        for c in range(n_ch):
            b = c & 1
            if c + 1 < n_ch:
                start(c + 1)
            cps[b].wait()
            pltpu.sync_copy(buf_v.at[b], out_hbm.at[pl.ds(base + c * ch, ch)])

    return k(table, gidx)


def _ln_rows(x, g, b):
    mu = jnp.mean(x, -1, keepdims=True)
    var = jnp.mean((x - mu) ** 2, -1, keepdims=True)
    return (x - mu) / jnp.sqrt(var + 1e-5) * g + b


def _pass2_body(he_ref, g_ref, ctra_ref, z1_ref, hv_ref,
                exp_ref, trep_ref, ssum_ref,
                w1he_ref, wv_ref, w1nn_ref,
                w2_ref, b2_ref, w3_ref, b3_ref,
                ln0g_ref, ln0b_ref, wd1_ref, bd1_ref, wd2_ref, bd2_ref,
                ln1g_ref, ln1b_ref, out_ref):
    f32 = jnp.float32
    he = he_ref[...]
    g = g_ref[...]
    ctrn = jnp.concatenate([ctra_ref[...], z1_ref[...]], axis=1)
    ab = jnp.dot(exp_ref[...], ctrn.astype(jnp.bfloat16),
                 preferred_element_type=f32)
    gu = lax.bitcast_convert_type(g[:, 0:64], jnp.uint32)
    r_hi = lax.bitcast_convert_type(gu & jnp.uint32(0xFFFF0000), f32)
    r_lo = lax.bitcast_convert_type(gu << 16, f32)
    rfull = jnp.concatenate([r_hi, r_lo], axis=1)
    npg = g[:, 64:88]
    v = (jnp.dot(npg, trep_ref[...], preferred_element_type=f32)
         * ab[:, 0:120] + ab[:, 120:240])
    ns = jnp.dot(v * v, ssum_ref[...], preferred_element_type=f32)
    nn = jnp.sqrt(ns + 1e-8)
    x1 = (ab[:, 240:368] + rfull
          + jnp.dot(he, w1he_ref[...], preferred_element_type=f32)
          + jnp.dot(v, wv_ref[...], preferred_element_type=f32)
          + jnp.dot(nn, w1nn_ref[...], preferred_element_type=f32))
    h1 = jnp.maximum(x1, 0.0).astype(jnp.bfloat16)
    h2 = jnp.maximum(
        jnp.dot(h1, w2_ref[...], preferred_element_type=f32) + b2_ref[...],
        0.0).astype(jnp.bfloat16)
    h3 = jnp.dot(h2, w3_ref[...], preferred_element_type=f32) + b3_ref[...]
    m = jnp.mean(h3.reshape(_TN, _K, _H), axis=1)
    h = _ln_rows(hv_ref[...] + m, ln0g_ref[...], ln0b_ref[...])
    dm = jnp.dot(
        jnp.maximum(jnp.dot(h, wd1_ref[...],
                            preferred_element_type=f32) + bd1_ref[...], 0.0),
        wd2_ref[...], preferred_element_type=f32) + bd2_ref[...]
    out_ref[...] = _ln_rows(h + dm, ln1g_ref[...], ln1b_ref[...])


def _pass2(he_r, g, ctra, z1, hv2, expm, trep, ssum, w1he, wv, w1nn,
           w2, b2, w3, b3, ln0g, ln0b, wd1, bd1, wd2, bd2, ln1g, ln1b,
           base_node, n_nodes):
    grid = (n_nodes // _TN,)
    boff = base_node // _TN
    erow = lambda i: (i + boff, 0)
    nrow = lambda i: (i + boff, 0)
    gorow = lambda i: (i, 0)
    whole = lambda i: (0, 0)
    return pl.pallas_call(
        _pass2_body,
        grid=grid,
        in_specs=[
            pl.BlockSpec((_E2, _D), erow),
            pl.BlockSpec((_E2, _TW), gorow),
            pl.BlockSpec((_TN, _CW), nrow),
            pl.BlockSpec((_TN, _H), nrow),
            pl.BlockSpec((_TN, _D), nrow),
            pl.BlockSpec((_E2, _TN), whole),
            pl.BlockSpec((24, 120), whole),
            pl.BlockSpec((120, 16), whole),
            pl.BlockSpec((_D, _H), whole),
            pl.BlockSpec((120, _H), whole),
            pl.BlockSpec((16, _H), whole),
            pl.BlockSpec((_H, _H), whole),
            pl.BlockSpec((1, _H), whole),
            pl.BlockSpec((_H, _H), whole),
            pl.BlockSpec((1, _H), whole),
            pl.BlockSpec((1, _H), whole),
            pl.BlockSpec((1, _H), whole),
            pl.BlockSpec((_H, 4 * _H), whole),
            pl.BlockSpec((1, 4 * _H), whole),
            pl.BlockSpec((4 * _H, _H), whole),
            pl.BlockSpec((1, _H), whole),
            pl.BlockSpec((1, _H), whole),
            pl.BlockSpec((1, _H), whole),
        ],
        out_specs=pl.BlockSpec((_TN, _H), gorow),
        out_shape=jax.ShapeDtypeStruct((n_nodes, _H), jnp.float32),
    )(he_r, g, ctra, z1, hv2, expm, trep, ssum, w1he, wv, w1nn,
      w2, b2, w3, b3, ln0g, ln0b, wd1, bd1, wd2, bd2, ln1g, ln1b)


def kernel(h_V, h_E, E_idx, X, Wp, bp, W1, b1, W2, b2, W3, b3,
           ln0_g, ln0_b, Wd1, bd1, Wd2, bd2, ln1_g, ln1_b):
    f32 = jnp.float32
    cm = np.array([p * 3 + i for i in range(3) for p in range(_P)])
    wp_cm = Wp[:, cm]
    bp_cm = bp[cm].reshape(1, 24)
    w1c = W1[256:384]
    wnode = jnp.concatenate([W1[0:128], W1[384 + cm], W1[408:416]], axis=0)
    w1nbl = W1[416 + cm]
    w1q = [jnp.concatenate([w1nbl[8 * i:8 * i + 8]] * 3, axis=0)
           for i in range(3)]
    z24 = jnp.zeros((24, _H), f32)
    wv = jnp.concatenate([z24, w1q[0], w1q[1], w1q[2], z24], axis=0)
    w1nn = jnp.concatenate([W1[440:448], W1[448:456]], axis=0)
    w1he = W1[128:256]
    expm = jnp.asarray(np.kron(np.eye(_TN, dtype=np.float32),
                               np.ones((_K, 1), np.float32))
                       ).astype(jnp.bfloat16)
    trep = jnp.asarray(np.tile(np.eye(24, dtype=np.float32), (1, 5)))
    ssum_np = np.zeros((120, 16), np.float32)
    for j in range(3):
        for p in range(8):
            ssum_np[8 * j + p, p] = 1.0
            ssum_np[96 + 8 * j + p, 8 + p] = 1.0
    ssum = jnp.asarray(ssum_np)

    x12 = X.reshape(_B * _N, 12)
    hv2 = h_V.reshape(_B * _N, _D)
    table, ctra, z1 = _pass1(x12, hv2, wp_cm, bp_cm, w1c, wnode,
                             b1.reshape(1, _H), wv)

    gidx = (E_idx + (jnp.arange(_B, dtype=jnp.int32) * _N)[:, None, None]
            ).reshape(-1)
    he_r = h_E.reshape(_B * _N * _K, _D)
    n_chunks = 2
    m = _B * _N
    cn = m // n_chunks
    gs = [_sc_gather(table, lax.slice(gidx, (c * cn * _K,),
                                      ((c + 1) * cn * _K,)))
          for c in range(n_chunks)]
    outs = []
    for c in range(n_chunks):
        outs.append(_pass2(
            he_r, gs[c], ctra, z1, hv2, expm, trep, ssum, w1he, wv, w1nn,
            W2.astype(jnp.bfloat16), b2.reshape(1, _H),
            W3.astype(jnp.bfloat16), b3.reshape(1, _H),
            ln0_g.reshape(1, _H), ln0_b.reshape(1, _H),
            Wd1, bd1.reshape(1, 4 * _H), Wd2, bd2.reshape(1, _H),
            ln1_g.reshape(1, _H), ln1_b.reshape(1, _H),
            base_node=c * cn, n_nodes=cn))
    out = jnp.concatenate(outs, axis=0)
    return out.reshape(_B, _N, _H), h_E


# speedup vs baseline: 1.1457x; 1.0057x over previous
import functools

import jax
import jax.numpy as jnp
import numpy as np
from jax import lax
from jax.experimental import pallas as pl
from jax.experimental.pallas import tpu as pltpu
from jax.experimental.pallas import tpu_sc as plsc

_B, _N, _K = 2, 1024, 32
_D, _H, _P = 128, 128, 8
_SCALE = 10.0
_TW = 128
_CW = 240
_T1 = 512
_TN = 128
_E2 = _TN * _K


def _pass1_body(x12_ref, hv_ref, wp_ref, bp_ref, w1c_ref, wnode_ref, b1_ref,
                wv_ref, table_ref, ctra_ref, z1_ref):
    x = x12_ref[...]
    hv = hv_ref[...]
    xn, xca, xc = x[:, 0:3], x[:, 3:6], x[:, 6:9]
    v1 = xc - xca
    v2 = xn - xca
    e1 = v1 / jnp.sqrt(jnp.sum(v1 * v1, -1, keepdims=True) + 1e-8)
    u2 = v2 - e1 * jnp.sum(e1 * v2, -1, keepdims=True)
    e2 = u2 / jnp.sqrt(jnp.sum(u2 * u2, -1, keepdims=True) + 1e-8)
    a1, a2, a3 = e1[:, 0:1], e1[:, 1:2], e1[:, 2:3]
    c1, c2, c3 = e2[:, 0:1], e2[:, 1:2], e2[:, 2:3]
    e3 = jnp.concatenate([a2 * c3 - a3 * c2, a3 * c1 - a1 * c3,
                          a1 * c2 - a2 * c1], axis=1)
    t = xca / _SCALE
    frames = jnp.concatenate([e1, e2, e3, t], axis=1)
    pl_cm = jnp.dot(hv, wp_ref[...],
                    preferred_element_type=jnp.float32) + bp_ref[...]
    plno = jnp.sqrt(pl_cm[:, 0:8] ** 2 + pl_cm[:, 8:16] ** 2
                    + pl_cm[:, 16:24] ** 2 + 1e-8)
    pgs = []
    for i in range(3):
        s = (frames[:, 0 + i:1 + i] * pl_cm[:, 0:8]
             + frames[:, 3 + i:4 + i] * pl_cm[:, 8:16]
             + frames[:, 6 + i:7 + i] * pl_cm[:, 16:24]
             + frames[:, 9 + i:10 + i])
        pgs.append(s)
    r = jnp.dot(hv, w1c_ref[...], preferred_element_type=jnp.float32)
    ru = lax.bitcast_convert_type(r, jnp.uint32) + jnp.uint32(0x8000)
    hi = ru[:, 0:64] & jnp.uint32(0xFFFF0000)
    lo = (ru[:, 64:128] & jnp.uint32(0xFFFF0000)) >> 16
    rp = lax.bitcast_convert_type(hi | lo, jnp.float32)
    pad = jnp.zeros((x.shape[0], _TW - 88), jnp.float32)
    table_ref[...] = jnp.concatenate([rp] + pgs + [pad], axis=1)
    ones8 = jnp.ones((1, 8), jnp.float32)
    tb = jnp.concatenate([frames[:, 9 + j:10 + j] * ones8 for j in range(3)],
                         axis=1)
    cblk = [jnp.concatenate(
        [frames[:, 3 * i + j:3 * i + j + 1] * ones8 for j in range(3)],
        axis=1) for i in range(3)]
    onesb = jnp.ones_like(tb)
    a_arr = jnp.concatenate([onesb, cblk[0], cblk[1], cblk[2], -onesb], axis=1)
    b_arr = jnp.concatenate(
        [-tb, -tb * cblk[0], -tb * cblk[1], -tb * cblk[2],
         jnp.concatenate(pgs, axis=1)], axis=1)
    ctra_ref[...] = jnp.concatenate([a_arr, b_arr], axis=1)
    del wv_ref
    z1_ref[...] = jnp.dot(jnp.concatenate([hv, pl_cm, plno], axis=1),
                          wnode_ref[...],
                          preferred_element_type=jnp.float32) + b1_ref[...]


def _pass1(x12, hv2, wp_cm, bp_cm, w1c, wnode, b1, wv):
    m = _B * _N
    grid = (m // _T1,)
    row = lambda i: (i, 0)
    whole = lambda i: (0, 0)
    return pl.pallas_call(
        _pass1_body,
        grid=grid,
        in_specs=[
            pl.BlockSpec((_T1, 12), row),
            pl.BlockSpec((_T1, _D), row),
            pl.BlockSpec((_D, 24), whole),
            pl.BlockSpec((1, 24), whole),
            pl.BlockSpec((_D, _H), whole),
            pl.BlockSpec((_D + 32, _H), whole),
            pl.BlockSpec((1, _H), whole),
            pl.BlockSpec((120, _H), whole),
        ],
        out_specs=[
            pl.BlockSpec((_T1, _TW), row),
            pl.BlockSpec((_T1, _CW), row),
            pl.BlockSpec((_T1, _H), row),
        ],
        out_shape=[
            jax.ShapeDtypeStruct((m, _TW), jnp.float32),
            jax.ShapeDtypeStruct((m, _CW), jnp.float32),
            jax.ShapeDtypeStruct((m, _H), jnp.float32),
        ],
    )(x12, hv2, wp_cm, bp_cm, w1c, wnode, b1, wv)


def _sc_gather(table, gidx):
    info = plsc.get_sparse_core_info()
    nw = info.num_cores * info.num_subcores
    tot = gidx.shape[0]
    per_w = tot // nw
    ch = 128
    n_ch = per_w // ch
    mesh = plsc.VectorSubcoreMesh(core_axis_name="c", subcore_axis_name="s")

    @functools.partial(
        pl.kernel, mesh=mesh,
        out_type=jax.ShapeDtypeStruct((tot, _TW), jnp.float32),
        scratch_types=[
            pltpu.VMEM((per_w,), jnp.int32),
            pltpu.VMEM((2, ch, _TW), jnp.float32),
            pltpu.SemaphoreType.DMA,
            pltpu.SemaphoreType.DMA,
        ],
    )
    def k(table_hbm, idx_hbm, out_hbm, idx_v, buf_v, sem0, sem1):
        wid = lax.axis_index("s") * info.num_cores + lax.axis_index("c")
        base = wid * per_w
        pltpu.sync_copy(idx_hbm.at[pl.ds(base, per_w)], idx_v)
        sems = (sem0, sem1)
        cps = [None, None]

        def start(c):
            b = c & 1
            cps[b] = pltpu.async_copy(
                table_hbm.at[idx_v.at[pl.ds(c * ch, ch)]], buf_v.at[b],
                sems[b])

        start(0)
        for c in range(n_ch):
            b = c & 1
            if c + 1 < n_ch:
                start(c + 1)
            cps[b].wait()
            pltpu.sync_copy(buf_v.at[b], out_hbm.at[pl.ds(base + c * ch, ch)])

    return k(table, gidx)


def _ln_rows(x, g, b):
    mu = jnp.mean(x, -1, keepdims=True)
    var = jnp.mean((x - mu) ** 2, -1, keepdims=True)
    return (x - mu) / jnp.sqrt(var + 1e-5) * g + b


def _pass2_body(he_ref, g_ref, ctra_ref, z1_ref, hv_ref,
                exp_ref, trep_ref, ssum_ref,
                w1he_ref, wv_ref, w1nn_ref,
                w2_ref, b2_ref, w3_ref, b3_ref,
                ln0g_ref, ln0b_ref, wd1_ref, bd1_ref, wd2_ref, bd2_ref,
                ln1g_ref, ln1b_ref, out_ref):
    f32 = jnp.float32
    he = he_ref[...]
    g = g_ref[...]
    ctrn = jnp.concatenate([ctra_ref[...], z1_ref[...]], axis=1)
    ab = jnp.dot(exp_ref[...], ctrn.astype(jnp.bfloat16),
                 preferred_element_type=f32)
    gu = lax.bitcast_convert_type(g[:, 0:64], jnp.uint32)
    r_hi = lax.bitcast_convert_type(gu & jnp.uint32(0xFFFF0000), f32)
    r_lo = lax.bitcast_convert_type(gu << 16, f32)
    rfull = jnp.concatenate([r_hi, r_lo], axis=1)
    npg = g[:, 64:88]
    v = (jnp.dot(npg, trep_ref[...], preferred_element_type=f32)
         * ab[:, 0:120] + ab[:, 120:240])
    ns = jnp.dot(v * v, ssum_ref[...], preferred_element_type=f32)
    nn = jnp.sqrt(ns + 1e-8)
    x1 = (ab[:, 240:368] + rfull
          + jnp.dot(he.astype(jnp.bfloat16), w1he_ref[...],
                    preferred_element_type=f32)
          + jnp.dot(v.astype(jnp.bfloat16), wv_ref[...],
                    preferred_element_type=f32)
          + jnp.dot(nn, w1nn_ref[...], preferred_element_type=f32))
    h1 = jnp.maximum(x1, 0.0).astype(jnp.bfloat16)
    h2 = jnp.maximum(
        jnp.dot(h1, w2_ref[...], preferred_element_type=f32) + b2_ref[...],
        0.0).astype(jnp.bfloat16)
    h3 = jnp.dot(h2, w3_ref[...], preferred_element_type=f32) + b3_ref[...]
    m = jnp.mean(h3.reshape(_TN, _K, _H), axis=1)
    h = _ln_rows(hv_ref[...] + m, ln0g_ref[...], ln0b_ref[...])
    dm = jnp.dot(
        jnp.maximum(jnp.dot(h, wd1_ref[...],
                            preferred_element_type=f32) + bd1_ref[...], 0.0),
        wd2_ref[...], preferred_element_type=f32) + bd2_ref[...]
    out_ref[...] = _ln_rows(h + dm, ln1g_ref[...], ln1b_ref[...])


def _pass2(he_r, g, ctra, z1, hv2, expm, trep, ssum, w1he, wv, w1nn,
           w2, b2, w3, b3, ln0g, ln0b, wd1, bd1, wd2, bd2, ln1g, ln1b,
           base_node, n_nodes):
    grid = (n_nodes // _TN,)
    boff = base_node // _TN
    erow = lambda i: (i + boff, 0)
    nrow = lambda i: (i + boff, 0)
    gorow = lambda i: (i, 0)
    whole = lambda i: (0, 0)
    return pl.pallas_call(
        _pass2_body,
        grid=grid,
        in_specs=[
            pl.BlockSpec((_E2, _D), erow),
            pl.BlockSpec((_E2, _TW), gorow),
            pl.BlockSpec((_TN, _CW), nrow),
            pl.BlockSpec((_TN, _H), nrow),
            pl.BlockSpec((_TN, _D), nrow),
            pl.BlockSpec((_E2, _TN), whole),
            pl.BlockSpec((24, 120), whole),
            pl.BlockSpec((120, 16), whole),
            pl.BlockSpec((_D, _H), whole),
            pl.BlockSpec((120, _H), whole),
            pl.BlockSpec((16, _H), whole),
            pl.BlockSpec((_H, _H), whole),
            pl.BlockSpec((1, _H), whole),
            pl.BlockSpec((_H, _H), whole),
            pl.BlockSpec((1, _H), whole),
            pl.BlockSpec((1, _H), whole),
            pl.BlockSpec((1, _H), whole),
            pl.BlockSpec((_H, 4 * _H), whole),
            pl.BlockSpec((1, 4 * _H), whole),
            pl.BlockSpec((4 * _H, _H), whole),
            pl.BlockSpec((1, _H), whole),
            pl.BlockSpec((1, _H), whole),
            pl.BlockSpec((1, _H), whole),
        ],
        out_specs=pl.BlockSpec((_TN, _H), gorow),
        out_shape=jax.ShapeDtypeStruct((n_nodes, _H), jnp.float32),
    )(he_r, g, ctra, z1, hv2, expm, trep, ssum, w1he, wv, w1nn,
      w2, b2, w3, b3, ln0g, ln0b, wd1, bd1, wd2, bd2, ln1g, ln1b)


def kernel(h_V, h_E, E_idx, X, Wp, bp, W1, b1, W2, b2, W3, b3,
           ln0_g, ln0_b, Wd1, bd1, Wd2, bd2, ln1_g, ln1_b):
    f32 = jnp.float32
    cm = np.array([p * 3 + i for i in range(3) for p in range(_P)])
    wp_cm = Wp[:, cm]
    bp_cm = bp[cm].reshape(1, 24)
    w1c = W1[256:384]
    wnode = jnp.concatenate([W1[0:128], W1[384 + cm], W1[408:416]], axis=0)
    w1nbl = W1[416 + cm]
    w1q = [jnp.concatenate([w1nbl[8 * i:8 * i + 8]] * 3, axis=0)
           for i in range(3)]
    z24 = jnp.zeros((24, _H), f32)
    wv = jnp.concatenate([z24, w1q[0], w1q[1], w1q[2], z24], axis=0)
    w1nn = jnp.concatenate([W1[440:448], W1[448:456]], axis=0)
    w1he = W1[128:256]
    wv_b = wv.astype(jnp.bfloat16)
    w1he_b = w1he.astype(jnp.bfloat16)
    expm = jnp.asarray(np.kron(np.eye(_TN, dtype=np.float32),
                               np.ones((_K, 1), np.float32))
                       ).astype(jnp.bfloat16)
    trep = jnp.asarray(np.tile(np.eye(24, dtype=np.float32), (1, 5)))
    ssum_np = np.zeros((120, 16), np.float32)
    for j in range(3):
        for p in range(8):
            ssum_np[8 * j + p, p] = 1.0
            ssum_np[96 + 8 * j + p, 8 + p] = 1.0
    ssum = jnp.asarray(ssum_np)

    x12 = X.reshape(_B * _N, 12)
    hv2 = h_V.reshape(_B * _N, _D)
    table, ctra, z1 = _pass1(x12, hv2, wp_cm, bp_cm, w1c, wnode,
                             b1.reshape(1, _H), wv)

    gidx = (E_idx + (jnp.arange(_B, dtype=jnp.int32) * _N)[:, None, None]
            ).reshape(-1)
    he_r = h_E.reshape(_B * _N * _K, _D)
    n_chunks = 2
    m = _B * _N
    cn = m // n_chunks
    gs = [_sc_gather(table, lax.slice(gidx, (c * cn * _K,),
                                      ((c + 1) * cn * _K,)))
          for c in range(n_chunks)]
    outs = []
    for c in range(n_chunks):
        outs.append(_pass2(
            he_r, gs[c], ctra, z1, hv2, expm, trep, ssum, w1he_b, wv_b, w1nn,
            W2.astype(jnp.bfloat16), b2.reshape(1, _H),
            W3.astype(jnp.bfloat16), b3.reshape(1, _H),
            ln0_g.reshape(1, _H), ln0_b.reshape(1, _H),
            Wd1, bd1.reshape(1, 4 * _H), Wd2, bd2.reshape(1, _H),
            ln1_g.reshape(1, _H), ln1_b.reshape(1, _H),
            base_node=c * cn, n_nodes=cn))
    out = jnp.concatenate(outs, axis=0)
    return out.reshape(_B, _N, _H), h_E
